# baseline (speedup 1.0000x reference)
import functools

import numpy as np
import jax
import jax.numpy as jnp
from jax import lax
from jax.experimental import pallas as pl
from jax.experimental.pallas import tpu as pltpu
from jax.experimental.pallas import tpu_sc as plsc

_N_EDGES = 3200000
_DROP_P = 0.2
_B = 512
_NW = 32
_D = 16


def _kept_indices() -> np.ndarray:
    key = jax.random.fold_in(jax.random.key(0), 12345)
    keep = np.asarray(jax.random.uniform(key, (_N_EDGES,)) >= _DROP_P)
    return np.nonzero(keep)[0].astype(np.int32)


_IDX = _kept_indices()
_K = int(_IDX.size)
_NSC = _K // _B
_T = _K - _NSC * _B
_TAIL_OFF = _NSC * _B
_SPW = -(-_NSC // _NW)
_TGROUPS = -(-_T // 16)

_stars = _IDX[: _NSC * _B : _B].astype(np.int64)
_ends = _IDX[_B - 1 :: _B][: _NSC].astype(np.int64)
_span = int((_ends - _stars).max()) + 1
_span = max(_span, int(_IDX[-1]) - int(_IDX[_TAIL_OFF]) + 1)
_W = ((_span + 7 + 7) // 8) * 8


def _padded_idx() -> np.ndarray:
    pad = np.full(((_NSC + 1) * _B,), _IDX[-1], np.int32)
    pad[:_K] = _IDX
    return pad


_IDX_ARR = jnp.asarray(_padded_idx())

_mesh = plsc.VectorSubcoreMesh(core_axis_name="c", subcore_axis_name="s")


@functools.partial(
    pl.kernel,
    out_type=(
        jax.ShapeDtypeStruct((_K,), jnp.int32),
        jax.ShapeDtypeStruct((_K,), jnp.int32),
        jax.ShapeDtypeStruct((_D, _K), jnp.float32),
    ),
    mesh=_mesh,
    scratch_types=[
        [pltpu.VMEM((_B,), jnp.int32)] * 2,
        [pltpu.VMEM((_D, _W), jnp.float32)] * 2,
        [pltpu.VMEM((_W,), jnp.int32)] * 2,
        [pltpu.VMEM((_W,), jnp.int32)] * 2,
        [pltpu.VMEM((_D, _B), jnp.float32)] * 2,
        [pltpu.VMEM((_B,), jnp.int32)] * 2,
        [pltpu.VMEM((_B,), jnp.int32)] * 2,
        [pltpu.SemaphoreType.DMA] * 2,
        [pltpu.SemaphoreType.DMA] * 2,
        [pltpu.SemaphoreType.DMA] * 2,
    ],
    compiler_params=pltpu.CompilerParams(
        use_tc_tiling_on_sc=False, needs_layout_passes=False),
)
def _sc_compact(idx_hbm, e0_hbm, e1_hbm, attr_t_hbm,
                out_e0_hbm, out_e1_hbm, out_attr_t_hbm,
                idx_v, wa_v, w0_v, w1_v, oa_v, o0_v, o1_v,
                sem_i, sem_w, sem_o):
    wid = lax.axis_index("s") * 2 + lax.axis_index("c")
    s0 = wid * _SPW
    s1 = jnp.minimum(s0 + _SPW, _NSC)
    my_n = s1 - s0

    def stage(g, b):
        return pltpu.async_copy(
            idx_hbm.at[pl.ds(pl.multiple_of(g * _B, _B), _B)], idx_v[b],
            sem_i[b])

    def drain_idx(b):
        pltpu.make_async_copy(
            idx_hbm.at[pl.ds(0, _B)], idx_v[b], sem_i[b]).wait()

    def window_base(b):
        head = idx_v[b][pl.ds(0, 16)]
        win = (head[0] // 8) * 8
        win = jnp.minimum(win, _N_EDGES - _W)
        return pl.multiple_of(win, 8)

    def fire_windows(b, win):
        return [
            pltpu.async_copy(attr_t_hbm.at[:, pl.ds(win, _W)], wa_v[b],
                             sem_w[b]),
            pltpu.async_copy(e0_hbm.at[pl.ds(win, _W)], w0_v[b], sem_w[b]),
            pltpu.async_copy(e1_hbm.at[pl.ds(win, _W)], w1_v[b], sem_w[b]),
        ]

    def compact(b, win, ngroups):
        def body(i, carry):
            sl = pl.ds(i * 16, 16)
            loc = idx_v[b][sl] - win
            o0_v[b][sl] = plsc.load_gather(w0_v[b], [loc])
            o1_v[b][sl] = plsc.load_gather(w1_v[b], [loc])
            for d in range(_D):
                oa_v[b][d, sl] = plsc.load_gather(wa_v[b].at[d], [loc])
            return carry
        lax.fori_loop(0, ngroups, body, 0)

    def fire_outs(g, b):
        off = pl.multiple_of(g * _B, _B)
        return [
            pltpu.async_copy(oa_v[b], out_attr_t_hbm.at[:, pl.ds(off, _B)],
                             sem_o[b]),
            pltpu.async_copy(o0_v[b], out_e0_hbm.at[pl.ds(off, _B)],
                             sem_o[b]),
            pltpu.async_copy(o1_v[b], out_e1_hbm.at[pl.ds(off, _B)],
                             sem_o[b]),
        ]

    def wait_all(cps):
        for cp in cps:
            cp.wait()

    @pl.when(my_n > 0)
    def _p0():
        stage(s0, 0)

    @pl.when(my_n > 1)
    def _p1():
        stage(s0 + 1, 1)

    def pipe_body(k, carry):
        g0 = s0 + 2 * k
        g1 = g0 + 1

        @pl.when(g0 < s1)
        def _b0():
            drain_idx(0)
            win0 = window_base(0)
            cpw0 = fire_windows(0, win0)

            @pl.when(g1 < s1)
            def _pre1():
                drain_idx(1)

            wait_all(cpw0)
            compact(0, win0, _B // 16)
            @pl.when(g0 + 2 < s1)
            def _():
                stage(g0 + 2, 0)
            co0 = fire_outs(g0, 0)

            @pl.when(g1 < s1)
            def _b1():
                win1 = window_base(1)
                cpw1 = fire_windows(1, win1)
                wait_all(cpw1)
                compact(1, win1, _B // 16)
                @pl.when(g1 + 2 < s1)
                def _():
                    stage(g1 + 2, 1)
                wait_all(fire_outs(g1, 1))

            wait_all(co0)

        return carry

    lax.fori_loop(0, (_SPW + 1) // 2, pipe_body, 0)

    @pl.when(wid == _NW - 1)
    def _tail():
        pltpu.sync_copy(
            idx_hbm.at[pl.ds(_TAIL_OFF, _B)], idx_v[0])
        win = window_base(0)
        wait_all(fire_windows(0, win))
        compact(0, win, _TGROUPS)
        for d in range(_D):
            pltpu.sync_copy(oa_v[0].at[d].at[pl.ds(0, _T)],
                            out_attr_t_hbm.at[d].at[pl.ds(_TAIL_OFF, _T)])
        pltpu.sync_copy(o0_v[0].at[pl.ds(0, _T)],
                        out_e0_hbm.at[pl.ds(_TAIL_OFF, _T)])
        pltpu.sync_copy(o1_v[0].at[pl.ds(0, _T)],
                        out_e1_hbm.at[pl.ds(_TAIL_OFF, _T)])


def kernel(edge_index, edge_attr):
    out_r0, out_r1, out_attr_t = _sc_compact(
        _IDX_ARR, edge_index[0], edge_index[1], edge_attr.T)
    return jnp.stack([out_r0, out_r1]), out_attr_t.T


# speedup vs baseline: 2.9290x; 2.9290x over previous
import functools

import numpy as np
import jax
import jax.numpy as jnp
from jax import lax
from jax.experimental import pallas as pl
from jax.experimental.pallas import tpu as pltpu
from jax.experimental.pallas import tpu_sc as plsc

_N_EDGES = 3200000
_DROP_P = 0.2
_B = 512
_NW = 32
_D = 16


def _kept_indices() -> np.ndarray:
    key = jax.random.fold_in(jax.random.key(0), 12345)
    keep = np.asarray(jax.random.uniform(key, (_N_EDGES,)) >= _DROP_P)
    return np.nonzero(keep)[0].astype(np.int32)


_IDX = _kept_indices()
_K = int(_IDX.size)
_NSC = _K // _B
_T = _K - _NSC * _B
_TAIL_OFF = _NSC * _B
_SPW = -(-_NSC // _NW)
_TGROUPS = -(-_T // 16)

_stars = _IDX[: _NSC * _B : _B].astype(np.int64)
_ends = _IDX[_B - 1 :: _B][: _NSC].astype(np.int64)
_span = int((_ends - _stars).max()) + 1
_span = max(_span, int(_IDX[-1]) - int(_IDX[_TAIL_OFF]) + 1)
_W = ((_span + 7 + 7) // 8) * 8


def _padded_idx() -> np.ndarray:
    pad = np.full(((_NSC + 1) * _B,), _IDX[-1], np.int32)
    pad[:_K] = _IDX
    return pad


_IDX_ARR = jnp.asarray(_padded_idx())

_mesh = plsc.VectorSubcoreMesh(core_axis_name="c", subcore_axis_name="s")


@functools.partial(
    pl.kernel,
    out_type=(
        jax.ShapeDtypeStruct((_K,), jnp.int32),
        jax.ShapeDtypeStruct((_K,), jnp.int32),
        tuple(jax.ShapeDtypeStruct((_K,), jnp.float32) for _ in range(_D)),
    ),
    mesh=_mesh,
    scratch_types=[
        [pltpu.VMEM((_B,), jnp.int32)] * 2,
        [pltpu.VMEM((_D, _W), jnp.float32)] * 2,
        [pltpu.VMEM((_W,), jnp.int32)] * 2,
        [pltpu.VMEM((_W,), jnp.int32)] * 2,
        [pltpu.VMEM((_D, _B), jnp.float32)] * 2,
        [pltpu.VMEM((_B,), jnp.int32)] * 2,
        [pltpu.VMEM((_B,), jnp.int32)] * 2,
        [pltpu.SemaphoreType.DMA] * 2,
        [pltpu.SemaphoreType.DMA] * 2,
        [pltpu.SemaphoreType.DMA] * 2,
    ],
    compiler_params=pltpu.CompilerParams(
        use_tc_tiling_on_sc=False, needs_layout_passes=False),
)
def _sc_compact(idx_hbm, e0_hbm, e1_hbm, attr_t_hbm,
                out_e0_hbm, out_e1_hbm, out_attr_hbms,
                idx_v, wa_v, w0_v, w1_v, oa_v, o0_v, o1_v,
                sem_i, sem_w, sem_o):
    wid = lax.axis_index("s") * 2 + lax.axis_index("c")
    s0 = wid * _SPW
    s1 = jnp.minimum(s0 + _SPW, _NSC)
    my_n = s1 - s0

    def stage(g, b):
        return pltpu.async_copy(
            idx_hbm.at[pl.ds(pl.multiple_of(g * _B, _B), _B)], idx_v[b],
            sem_i[b])

    def drain_idx(b):
        pltpu.make_async_copy(
            idx_hbm.at[pl.ds(0, _B)], idx_v[b], sem_i[b]).wait()

    def window_base(b):
        head = idx_v[b][pl.ds(0, 16)]
        win = (head[0] // 8) * 8
        win = jnp.minimum(win, _N_EDGES - _W)
        return pl.multiple_of(win, 8)

    def fire_windows(b, win):
        return [
            pltpu.async_copy(attr_t_hbm.at[:, pl.ds(win, _W)], wa_v[b],
                             sem_w[b]),
            pltpu.async_copy(e0_hbm.at[pl.ds(win, _W)], w0_v[b], sem_w[b]),
            pltpu.async_copy(e1_hbm.at[pl.ds(win, _W)], w1_v[b], sem_w[b]),
        ]

    def compact(b, win, ngroups):
        def body(i, carry):
            sl = pl.ds(i * 16, 16)
            loc = idx_v[b][sl] - win
            o0_v[b][sl] = plsc.load_gather(w0_v[b], [loc])
            o1_v[b][sl] = plsc.load_gather(w1_v[b], [loc])
            for d in range(_D):
                oa_v[b][d, sl] = plsc.load_gather(wa_v[b].at[d], [loc])
            return carry
        lax.fori_loop(0, ngroups, body, 0)

    def fire_outs(g, b):
        off = pl.multiple_of(g * _B, _B)
        cps = [
            pltpu.async_copy(o0_v[b], out_e0_hbm.at[pl.ds(off, _B)],
                             sem_o[b]),
            pltpu.async_copy(o1_v[b], out_e1_hbm.at[pl.ds(off, _B)],
                             sem_o[b]),
        ]
        for d in range(_D):
            cps.append(pltpu.async_copy(
                oa_v[b].at[d], out_attr_hbms[d].at[pl.ds(off, _B)],
                sem_o[b]))
        return cps

    def wait_all(cps):
        for cp in cps:
            cp.wait()

    @pl.when(my_n > 0)
    def _p0():
        stage(s0, 0)

    @pl.when(my_n > 1)
    def _p1():
        stage(s0 + 1, 1)

    def pipe_body(k, carry):
        g0 = s0 + 2 * k
        g1 = g0 + 1

        @pl.when(g0 < s1)
        def _b0():
            drain_idx(0)
            win0 = window_base(0)
            cpw0 = fire_windows(0, win0)

            @pl.when(g1 < s1)
            def _pre1():
                drain_idx(1)

            wait_all(cpw0)
            compact(0, win0, _B // 16)
            @pl.when(g0 + 2 < s1)
            def _():
                stage(g0 + 2, 0)
            co0 = fire_outs(g0, 0)

            @pl.when(g1 < s1)
            def _b1():
                win1 = window_base(1)
                cpw1 = fire_windows(1, win1)
                wait_all(cpw1)
                compact(1, win1, _B // 16)
                @pl.when(g1 + 2 < s1)
                def _():
                    stage(g1 + 2, 1)
                wait_all(fire_outs(g1, 1))

            wait_all(co0)

        return carry

    lax.fori_loop(0, (_SPW + 1) // 2, pipe_body, 0)

    @pl.when(wid == _NW - 1)
    def _tail():
        pltpu.sync_copy(
            idx_hbm.at[pl.ds(_TAIL_OFF, _B)], idx_v[0])
        win = window_base(0)
        wait_all(fire_windows(0, win))
        compact(0, win, _TGROUPS)
        for d in range(_D):
            pltpu.sync_copy(oa_v[0].at[d].at[pl.ds(0, _T)],
                            out_attr_hbms[d].at[pl.ds(_TAIL_OFF, _T)])
        pltpu.sync_copy(o0_v[0].at[pl.ds(0, _T)],
                        out_e0_hbm.at[pl.ds(_TAIL_OFF, _T)])
        pltpu.sync_copy(o1_v[0].at[pl.ds(0, _T)],
                        out_e1_hbm.at[pl.ds(_TAIL_OFF, _T)])


def kernel(edge_index, edge_attr):
    out_r0, out_r1, out_cols = _sc_compact(
        _IDX_ARR, edge_index[0], edge_index[1], edge_attr.T)
    return jnp.stack([out_r0, out_r1]), jnp.stack(out_cols).T
